# baseline (speedup 1.0000x reference)
import functools

import jax
import jax.numpy as jnp
from jax import lax
from jax.experimental import pallas as pl
from jax.experimental.pallas import tpu as pltpu
from jax.experimental.pallas import tpu_sc as plsc

_F32 = jnp.float32
_I32 = jnp.int32


def _tc1_body(stk_ref, nf_hbm, wq_hbm, wn_hbm,
              aux_ref, rmean_ref,
              nf_v, wq_v, wn_v, sems):
    H = stk_ref.shape[1]
    B = stk_ref.shape[0] - 7
    N = nf_v.shape[0]
    MAX_PATHS = 4.0

    CH = 4
    R = H // CH

    def _start(hbm, vmem, base, row_splits):
        cps = []
        r0 = 0
        for i, nrows in enumerate(row_splits):
            cp = pltpu.make_async_copy(hbm.at[pl.ds(r0, nrows), :],
                                       vmem.at[pl.ds(r0, nrows), :],
                                       sems.at[base + i])
            cp.start()
            cps.append(cp)
            r0 += nrows
        return cps

    cp_wq = _start(wq_hbm, wq_v, 0, [R] * CH)
    cp_nf = _start(nf_hbm, nf_v, CH, [104, 96])
    cp_wn = _start(wn_hbm, wn_v, CH + 2, [R] * CH)

    q = stk_ref[0:B, :]
    bq = stk_ref[B:B + 1, :]
    bn = stk_ref[B + 1:B + 2, :]
    bidx_f = stk_ref[B + 6:B + 7, :N]

    iota_bf = jax.lax.broadcasted_iota(_I32, (B, N), 0).astype(_F32)
    Mb = bidx_f == iota_bf
    Mf = Mb.astype(_F32)
    cnt = jnp.sum(Mf, axis=1, keepdims=True)
    iota_n = jax.lax.broadcasted_iota(_I32, (B, N), 1)
    k = jnp.minimum(cnt, 3.0)
    per = jnp.floor(MAX_PATHS / jnp.maximum(k, 1.0))

    def _dot_t(x, w):
        return jax.lax.dot_general(x, w, (((1,), (1,)), ((), ())),
                                   preferred_element_type=_F32)

    for cp in cp_wq:
        cp.wait()
    qp = _dot_t(q, wq_v[...]) + bq
    an = jnp.maximum(jnp.sqrt(jnp.sum(qp * qp, axis=1, keepdims=True)), 1e-8)

    for cp in cp_nf + cp_wn:
        cp.wait()
    npj = _dot_t(nf_v[...], wn_v[...]) + bn
    Bn = jnp.maximum(jnp.sqrt(jnp.sum(npj * npj, axis=1, keepdims=True)), 1e-8)

    S = jax.lax.dot_general(qp, npj, (((1,), (1,)), ((), ())),
                            preferred_element_type=_F32)
    S = S / (an * Bn.reshape(1, N))

    NEG = jnp.float32(-2.0)
    Ssel = jnp.where(Mb, S, NEG)
    gs = []
    for _ in range(3):
        m = jnp.max(Ssel, axis=1, keepdims=True)
        g = jnp.min(jnp.where(Ssel == m, iota_n, N), axis=1, keepdims=True)
        gs.append(g)
        Ssel = jnp.where(iota_n == g, NEG, Ssel)

    rmean_ref[...] = jnp.dot(Mf, npj, preferred_element_type=_F32) / jnp.maximum(cnt, 1.0)

    aux = jnp.concatenate(
        [k.astype(_I32), per.astype(_I32), gs[0], gs[1], gs[2],
         cnt.astype(_I32), jnp.zeros((B, 10), _I32)], axis=1)
    aux_ref[...] = aux


def _sc_body(ei_hbm, bidx_hbm, aux_hbm, u_hbm, np_hbm,
             src_v, dst_v, bidx_v, aux_v, u_v, np_v):
    cid = lax.axis_index("c")
    sid = lax.axis_index("s")

    @pl.when((cid == 0) & (sid == 0))
    def _work():
        pltpu.sync_copy(ei_hbm.at[0], src_v)
        pltpu.sync_copy(ei_hbm.at[1], dst_v)
        pltpu.sync_copy(bidx_hbm, bidx_v)
        pltpu.sync_copy(aux_hbm, aux_v)

        zero16 = jnp.zeros((16,), _F32)
        iota16 = lax.iota(_I32, 16)
        row16 = iota16 * 16

        def _zbody(i, carry):
            u_v[pl.ds(i * 16, 16)] = zero16
            return carry

        lax.fori_loop(0, u_v.shape[0] // 16, _zbody, 0)

        k_v = plsc.load_gather(aux_v, [row16])
        per_v = plsc.load_gather(aux_v, [row16 + 1])
        g0 = plsc.load_gather(aux_v, [row16 + 2])
        g1 = plsc.load_gather(aux_v, [row16 + 3])
        g2 = plsc.load_gather(aux_v, [row16 + 4])

        def _body(ci, carry):
            c0, c1, c2 = carry
            src16 = src_v[pl.ds(ci * 16, 16)]
            dst16 = dst_v[pl.ds(ci * 16, 16)]
            bd16 = plsc.load_gather(bidx_v, [dst16])
            for j in range(16):
                s_e = src16[j]
                d_e = dst16[j]
                in_b = iota16 == bd16[j]
                m0 = (g0 == s_e) & in_b & (k_v > 0) & (c0 < per_v)
                m1 = (g1 == s_e) & in_b & (k_v > 1) & (c1 < per_v)
                m2 = (g2 == s_e) & in_b & (k_v > 2) & (c2 < per_v)
                c0 = c0 + jnp.where(m0, 1, 0)
                c1 = c1 + jnp.where(m1, 1, 0)
                c2 = c2 + jnp.where(m2, 1, 0)
                take = jnp.where(m0 | m1 | m2, 1.0, 0.0)
                plsc.addupdate_scatter(u_v, [iota16 + d_e * 16], take)
            return c0, c1, c2

        c0, c1, c2 = lax.fori_loop(
            0, src_v.shape[0] // 16, _body,
            (jnp.zeros((16,), _I32), jnp.zeros((16,), _I32),
             jnp.zeros((16,), _I32)))

        plsc.addupdate_scatter(u_v, [g0 * 16 + iota16], c0.astype(_F32))
        plsc.addupdate_scatter(u_v, [g1 * 16 + iota16], c1.astype(_F32))
        plsc.addupdate_scatter(u_v, [g2 * 16 + iota16], c2.astype(_F32))

        np_v[...] = c0 + c1 + c2
        pltpu.sync_copy(u_v, u_hbm)
        pltpu.sync_copy(np_v, np_hbm)


def _tc2_body(stk_ref, nf_hbm, w1_hbm, w2_hbm, u_ref, np_ref, aux_ref,
              rmean_ref, out_ref, nf_v, w1_v, w2_v, sems):
    H = stk_ref.shape[1]
    B = stk_ref.shape[0] - 7
    N = nf_v.shape[0]

    CH = 4
    R = H // CH

    def _start(hbm, vmem, base, row_splits):
        cps = []
        r0 = 0
        for i, nrows in enumerate(row_splits):
            cp = pltpu.make_async_copy(hbm.at[pl.ds(r0, nrows), :],
                                       vmem.at[pl.ds(r0, nrows), :],
                                       sems.at[base + i])
            cp.start()
            cps.append(cp)
            r0 += nrows
        return cps

    cp_nf = _start(nf_hbm, nf_v, 0, [104, 96])
    cp_w1 = _start(w1_hbm, w1_v, 2, [R] * CH)
    cp_w2 = _start(w2_hbm, w2_v, 2 + CH, [R] * CH)

    b1 = stk_ref[B + 2:B + 3, :]
    b2 = stk_ref[B + 3:B + 4, :]
    gamma = stk_ref[B + 4:B + 5, :]
    beta = stk_ref[B + 5:B + 6, :]

    npaths = np_ref[...].astype(_F32)
    cnt = aux_ref[:, 5:6].astype(_F32)

    def _dot_t(x, w):
        return jax.lax.dot_general(x, w, (((1,), (1,)), ((), ())),
                                   preferred_element_type=_F32)

    for cp in cp_nf:
        cp.wait()
    aggsum = jax.lax.dot_general(u_ref[...], nf_v[...], (((0,), (0,)), ((), ())),
                                 preferred_element_type=_F32)
    agg = aggsum / (2.0 * jnp.maximum(npaths, 1.0))

    for cp in cp_w1:
        cp.wait()
    h = jnp.maximum(_dot_t(agg, w1_v[...]) + b1, 0.0)
    for cp in cp_w2:
        cp.wait()
    rep_paths = _dot_t(h, w2_v[...]) + b2

    rep = jnp.where(npaths > 0.0, rep_paths, rmean_ref[...])
    rep = jnp.where(cnt > 0.0, rep, 0.0)

    mu = jnp.mean(rep, axis=1, keepdims=True)
    d = rep - mu
    var = jnp.mean(d * d, axis=1, keepdims=True)
    out_ref[...] = d * jax.lax.rsqrt(var + 1e-5) * gamma + beta


@jax.jit
def _run(query, node_features, edge_index, batch_indices, Wq, bq, Wn, bn,
         W1, b1, W2, b2, gamma, beta):
    B, H = query.shape
    N = node_features.shape[0]
    E = edge_index.shape[1]
    bidx_pad = jnp.pad(batch_indices.astype(_F32), (0, H - N),
                       constant_values=B)
    stk = jnp.concatenate(
        [query, jnp.stack([bq, bn, b1, b2, gamma, beta]),
         bidx_pad.reshape(1, H)], axis=0)
    ei = edge_index.astype(_I32)
    hbm = pl.BlockSpec(memory_space=pl.ANY)
    auto = pl.BlockSpec()

    aux, rmean = pl.pallas_call(
        _tc1_body,
        out_shape=(jax.ShapeDtypeStruct((B, 16), _I32),
                   jax.ShapeDtypeStruct((B, H), _F32)),
        in_specs=[auto, hbm, hbm, hbm],
        scratch_shapes=[
            pltpu.VMEM((N, H), _F32), pltpu.VMEM((H, H), _F32),
            pltpu.VMEM((H, H), _F32),
            pltpu.SemaphoreType.DMA((10,)),
        ],
    )(stk, node_features, Wq, Wn)

    mesh = plsc.VectorSubcoreMesh(core_axis_name="c", subcore_axis_name="s")
    sc = functools.partial(
        pl.kernel, mesh=mesh,
        compiler_params=pltpu.CompilerParams(needs_layout_passes=False),
        out_type=(jax.ShapeDtypeStruct((N * 16,), _F32),
                  jax.ShapeDtypeStruct((16,), _I32)),
        scratch_types=[
            pltpu.VMEM((E,), _I32), pltpu.VMEM((E,), _I32),
            pltpu.VMEM((N,), _I32), pltpu.VMEM((B * 16,), _I32),
            pltpu.VMEM((N * 16,), _F32), pltpu.VMEM((16,), _I32),
        ],
    )(_sc_body)
    u_flat, npaths = sc(ei, batch_indices.astype(_I32), aux.reshape(B * 16))
    u = u_flat.reshape(N, 16)

    out = pl.pallas_call(
        _tc2_body,
        out_shape=jax.ShapeDtypeStruct((B, H), _F32),
        in_specs=[auto, hbm, hbm, hbm, auto, auto, auto, auto],
        scratch_shapes=[
            pltpu.VMEM((N, H), _F32), pltpu.VMEM((H, H), _F32),
            pltpu.VMEM((H, H), _F32),
            pltpu.SemaphoreType.DMA((10,)),
        ],
    )(stk, node_features, W1, W2, u, npaths.reshape(16, 1), aux, rmean)
    return out


def kernel(query, node_features, edge_index, batch_indices, Wq, bq, Wn, bn,
           W1, b1, W2, b2, gamma, beta):
    return _run(query, node_features, edge_index, batch_indices,
                Wq, bq, Wn, bn, W1, b1, W2, b2, gamma, beta)


# speedup vs baseline: 1.0362x; 1.0362x over previous
import functools

import jax
import jax.numpy as jnp
from jax import lax
from jax.experimental import pallas as pl
from jax.experimental.pallas import tpu as pltpu
from jax.experimental.pallas import tpu_sc as plsc

_F32 = jnp.float32
_I32 = jnp.int32


def _tc1_body(stk_ref, nf_hbm, wq_hbm, wn_hbm,
              aux_ref, rmean_ref,
              nf_v, wq_v, wn_v, sems):
    H = stk_ref.shape[1]
    B = stk_ref.shape[0] - 7
    N = nf_v.shape[0]
    MAX_PATHS = 4.0

    CH = 4
    R = H // CH

    def _start(hbm, vmem, base, row_splits):
        cps = []
        r0 = 0
        for i, nrows in enumerate(row_splits):
            cp = pltpu.make_async_copy(hbm.at[pl.ds(r0, nrows), :],
                                       vmem.at[pl.ds(r0, nrows), :],
                                       sems.at[base + i])
            cp.start()
            cps.append(cp)
            r0 += nrows
        return cps

    cp_wq = _start(wq_hbm, wq_v, 0, [R] * CH)
    cp_nf = _start(nf_hbm, nf_v, CH, [104, 96])
    cp_wn = _start(wn_hbm, wn_v, CH + 2, [R] * CH)

    q = stk_ref[0:B, :]
    bq = stk_ref[B:B + 1, :]
    bn = stk_ref[B + 1:B + 2, :]
    bidx_f = stk_ref[B + 6:B + 7, :N]

    iota_bf = jax.lax.broadcasted_iota(_I32, (B, N), 0).astype(_F32)
    Mb = bidx_f == iota_bf
    Mf = Mb.astype(_F32)
    cnt = jnp.sum(Mf, axis=1, keepdims=True)
    iota_n = jax.lax.broadcasted_iota(_I32, (B, N), 1)
    k = jnp.minimum(cnt, 3.0)
    per = jnp.floor(MAX_PATHS / jnp.maximum(k, 1.0))

    def _dot_t(x, w):
        return jax.lax.dot_general(x, w, (((1,), (1,)), ((), ())),
                                   preferred_element_type=_F32)

    for cp in cp_wq:
        cp.wait()
    qp = _dot_t(q, wq_v[...]) + bq
    an = jnp.maximum(jnp.sqrt(jnp.sum(qp * qp, axis=1, keepdims=True)), 1e-8)

    for cp in cp_nf + cp_wn:
        cp.wait()
    npj = _dot_t(nf_v[...], wn_v[...]) + bn
    Bn = jnp.maximum(jnp.sqrt(jnp.sum(npj * npj, axis=1, keepdims=True)), 1e-8)

    S = jax.lax.dot_general(qp, npj, (((1,), (1,)), ((), ())),
                            preferred_element_type=_F32)
    S = S / (an * Bn.reshape(1, N))

    NEG = jnp.float32(-2.0)
    Ssel = jnp.where(Mb, S, NEG)
    gs = []
    for _ in range(3):
        m = jnp.max(Ssel, axis=1, keepdims=True)
        g = jnp.min(jnp.where(Ssel == m, iota_n, N), axis=1, keepdims=True)
        gs.append(g)
        Ssel = jnp.where(iota_n == g, NEG, Ssel)

    rmean_ref[...] = jnp.dot(Mf, npj, preferred_element_type=_F32) / jnp.maximum(cnt, 1.0)

    aux = jnp.concatenate(
        [k.astype(_I32), per.astype(_I32), gs[0], gs[1], gs[2],
         cnt.astype(_I32), jnp.zeros((B, 10), _I32)], axis=1)
    aux_ref[...] = aux


def _sc_body(ei_hbm, bidx_hbm, aux_hbm, out_hbm,
             src_v, dst_v, bidx_v, aux_v, out_v):
    cid = lax.axis_index("c")
    sid = lax.axis_index("s")

    @pl.when((cid == 0) & (sid == 0))
    def _work():
        pltpu.sync_copy(ei_hbm.at[0], src_v)
        pltpu.sync_copy(ei_hbm.at[1], dst_v)
        pltpu.sync_copy(bidx_hbm, bidx_v)
        pltpu.sync_copy(aux_hbm, aux_v)

        iota16 = lax.iota(_I32, 16)
        zeros_i = jnp.zeros((16,), _I32)
        row16 = iota16 * 16

        def _zbody(i, carry):
            out_v[pl.ds(i * 16, 16)] = zeros_i
            return carry

        lax.fori_loop(0, out_v.shape[0] // 16, _zbody, 0)

        k_v = plsc.load_gather(aux_v, [row16])
        per_v = plsc.load_gather(aux_v, [row16 + 1])
        g0 = plsc.load_gather(aux_v, [row16 + 2])
        g1 = plsc.load_gather(aux_v, [row16 + 3])
        g2 = plsc.load_gather(aux_v, [row16 + 4])

        def _body(ci, carry):
            c0, c1, c2, pos = carry
            src16 = src_v[pl.ds(ci * 16, 16)]
            dst16 = dst_v[pl.ds(ci * 16, 16)]
            bd16 = plsc.load_gather(bidx_v, [dst16])
            for j in range(16):
                s_e = src16[j]
                d_e = dst16[j]
                in_b = iota16 == bd16[j]
                m0 = (g0 == s_e) & in_b & (k_v > 0) & (c0 < per_v)
                m1 = (g1 == s_e) & in_b & (k_v > 1) & (c1 < per_v)
                m2 = (g2 == s_e) & in_b & (k_v > 2) & (c2 < per_v)
                take = m0 | m1 | m2

                @pl.when(jnp.any(take))
                def _record():
                    plsc.store_scatter(out_v, [row16 + pos], zeros_i + d_e,
                                       mask=take)

                c0 = c0 + jnp.where(m0, 1, 0)
                c1 = c1 + jnp.where(m1, 1, 0)
                c2 = c2 + jnp.where(m2, 1, 0)
                pos = pos + jnp.where(take, 1, 0)
            return c0, c1, c2, pos

        c0, c1, c2, pos = lax.fori_loop(
            0, src_v.shape[0] // 16, _body,
            (zeros_i, zeros_i, zeros_i, zeros_i))

        plsc.store_scatter(out_v, [row16 + 4], pos)
        plsc.store_scatter(out_v, [row16 + 5], c0)
        plsc.store_scatter(out_v, [row16 + 6], c1)
        plsc.store_scatter(out_v, [row16 + 7], c2)
        plsc.store_scatter(out_v, [row16 + 8], g0)
        plsc.store_scatter(out_v, [row16 + 9], g1)
        plsc.store_scatter(out_v, [row16 + 10], g2)
        pltpu.sync_copy(out_v, out_hbm)


def _tc2_body(stk_ref, nf_hbm, w1_hbm, w2_hbm, sel_ref,
              rmean_ref, out_ref, nf_v, w1_v, w2_v, sems):
    H = stk_ref.shape[1]
    B = stk_ref.shape[0] - 7
    N = nf_v.shape[0]

    CH = 4
    R = H // CH

    def _start(hbm, vmem, base, row_splits):
        cps = []
        r0 = 0
        for i, nrows in enumerate(row_splits):
            cp = pltpu.make_async_copy(hbm.at[pl.ds(r0, nrows), :],
                                       vmem.at[pl.ds(r0, nrows), :],
                                       sems.at[base + i])
            cp.start()
            cps.append(cp)
            r0 += nrows
        return cps

    cp_nf = _start(nf_hbm, nf_v, 0, [104, 96])
    cp_w1 = _start(w1_hbm, w1_v, 2, [R] * CH)
    cp_w2 = _start(w2_hbm, w2_v, 2 + CH, [R] * CH)

    b1 = stk_ref[B + 2:B + 3, :]
    b2 = stk_ref[B + 3:B + 4, :]
    gamma = stk_ref[B + 4:B + 5, :]
    beta = stk_ref[B + 5:B + 6, :]

    npaths_i = sel_ref[:, 4:5]
    npaths = npaths_i.astype(_F32)
    iota_n = jax.lax.broadcasted_iota(_I32, (B, N), 1)
    u = jnp.zeros((B, N), _F32)
    for s in range(4):
        d_s = sel_ref[:, s:s + 1]
        u = u + ((iota_n == d_s) & (s < npaths_i)).astype(_F32)
    for j in range(3):
        c_j = sel_ref[:, 5 + j:6 + j].astype(_F32)
        g_j = sel_ref[:, 8 + j:9 + j]
        u = u + c_j * (iota_n == g_j).astype(_F32)

    def _dot_t(x, w):
        return jax.lax.dot_general(x, w, (((1,), (1,)), ((), ())),
                                   preferred_element_type=_F32)

    for cp in cp_nf:
        cp.wait()
    aggsum = jnp.dot(u, nf_v[...], preferred_element_type=_F32)
    agg = aggsum / (2.0 * jnp.maximum(npaths, 1.0))

    for cp in cp_w1:
        cp.wait()
    h = jnp.maximum(_dot_t(agg, w1_v[...]) + b1, 0.0)
    for cp in cp_w2:
        cp.wait()
    rep_paths = _dot_t(h, w2_v[...]) + b2

    rep = jnp.where(npaths > 0.0, rep_paths, rmean_ref[...])

    mu = jnp.mean(rep, axis=1, keepdims=True)
    d = rep - mu
    var = jnp.mean(d * d, axis=1, keepdims=True)
    out_ref[...] = d * jax.lax.rsqrt(var + 1e-5) * gamma + beta


@jax.jit
def _run(query, node_features, edge_index, batch_indices, Wq, bq, Wn, bn,
         W1, b1, W2, b2, gamma, beta):
    B, H = query.shape
    N = node_features.shape[0]
    E = edge_index.shape[1]
    bidx_pad = jnp.pad(batch_indices.astype(_F32), (0, H - N),
                       constant_values=B)
    stk = jnp.concatenate(
        [query, jnp.stack([bq, bn, b1, b2, gamma, beta]),
         bidx_pad.reshape(1, H)], axis=0)
    ei = edge_index.astype(_I32)
    hbm = pl.BlockSpec(memory_space=pl.ANY)
    auto = pl.BlockSpec()

    aux, rmean = pl.pallas_call(
        _tc1_body,
        out_shape=(jax.ShapeDtypeStruct((B, 16), _I32),
                   jax.ShapeDtypeStruct((B, H), _F32)),
        in_specs=[auto, hbm, hbm, hbm],
        scratch_shapes=[
            pltpu.VMEM((N, H), _F32), pltpu.VMEM((H, H), _F32),
            pltpu.VMEM((H, H), _F32),
            pltpu.SemaphoreType.DMA((10,)),
        ],
    )(stk, node_features, Wq, Wn)

    mesh = plsc.VectorSubcoreMesh(core_axis_name="c", subcore_axis_name="s")
    sc = functools.partial(
        pl.kernel, mesh=mesh,
        compiler_params=pltpu.CompilerParams(needs_layout_passes=False),
        out_type=jax.ShapeDtypeStruct((B * 16,), _I32),
        scratch_types=[
            pltpu.VMEM((E,), _I32), pltpu.VMEM((E,), _I32),
            pltpu.VMEM((N,), _I32), pltpu.VMEM((B * 16,), _I32),
            pltpu.VMEM((B * 16,), _I32),
        ],
    )(_sc_body)
    sel = sc(ei, batch_indices.astype(_I32), aux.reshape(B * 16))

    out = pl.pallas_call(
        _tc2_body,
        out_shape=jax.ShapeDtypeStruct((B, H), _F32),
        in_specs=[auto, hbm, hbm, hbm, auto, auto],
        scratch_shapes=[
            pltpu.VMEM((N, H), _F32), pltpu.VMEM((H, H), _F32),
            pltpu.VMEM((H, H), _F32),
            pltpu.SemaphoreType.DMA((10,)),
        ],
    )(stk, node_features, W1, W2, sel.reshape(B, 16), rmean)
    return out


def kernel(query, node_features, edge_index, batch_indices, Wq, bq, Wn, bn,
           W1, b1, W2, b2, gamma, beta):
    return _run(query, node_features, edge_index, batch_indices,
                Wq, bq, Wn, bn, W1, b1, W2, b2, gamma, beta)
